# Initial kernel scaffold; baseline (speedup 1.0000x reference)
import jax
import jax.numpy as jnp
from jax.experimental import pallas as pl


def kernel(x, W_w, W_b, Uiou_w, Uiou_b, Uf_w, Uf_b):
    raise NotImplementedError("write your pallas kernel here")


# speedup vs baseline: 8.9299x; 8.9299x over previous
import jax
import jax.numpy as jnp
from jax.experimental import pallas as pl

N = 100000
H = 128
X = 128
LEAF_START = 50000
L16_START = 65535
P15 = 17233
P15_PAD = 17408
L15_LEAVES = L16_START - LEAF_START


def _init_body(x_ref, w_ref, b_ref, h_ref, c_ref):
    t = jnp.tanh(jnp.dot(x_ref[:], w_ref[:], preferred_element_type=jnp.float32)
                 + b_ref[:])
    h_ref[:] = t[:, :H]
    c_ref[:] = t[:, H:]


def _gate_body(hcat_ref, ccat_ref, uf_ref, ufb_ref, uio_ref, uiob_ref,
               h_ref, c_ref):
    hc = hcat_ref[:]
    f = jax.nn.sigmoid(jnp.dot(hc, uf_ref[:], preferred_element_type=jnp.float32)
                       + ufb_ref[:])
    iou = jnp.dot(hc, uio_ref[:], preferred_element_type=jnp.float32) + uiob_ref[:]
    i = jax.nn.sigmoid(iou[:, :H])
    o = jax.nn.sigmoid(iou[:, H:2 * H])
    u = jnp.tanh(iou[:, 2 * H:])
    fc = f * ccat_ref[:]
    c_new = i * u + fc[:, :H] + fc[:, H:]
    c_ref[:] = c_new
    h_ref[:] = o * jnp.tanh(c_new)


def _level_call(hcat, ccat, uf, ufb, uio, uiob):
    P = hcat.shape[0]
    if P % 2048 == 0:
        pb = min(P, 2048)
    elif P % 1024 == 0:
        pb = 1024
    else:
        pb = P
    grid = P // pb
    blk2h = pl.BlockSpec((pb, 2 * H), lambda i: (i, 0))
    blkh = pl.BlockSpec((pb, H), lambda i: (i, 0))
    full = lambda a: pl.BlockSpec(a.shape, lambda i: (0,) * a.ndim)
    return pl.pallas_call(
        _gate_body,
        grid=(grid,),
        in_specs=[blk2h, blk2h, full(uf), full(ufb), full(uio), full(uiob)],
        out_specs=[blkh, blkh],
        out_shape=[jax.ShapeDtypeStruct((P, H), jnp.float32),
                   jax.ShapeDtypeStruct((P, H), jnp.float32)],
    )(hcat, ccat, uf, ufb, uio, uiob)


def kernel(x, W_w, W_b, Uiou_w, Uiou_b, Uf_w, Uf_b):
    wb = W_b.reshape(1, 2 * H)
    ufb = Uf_b.reshape(1, 2 * H)
    uiob = Uiou_b.reshape(1, 3 * H)

    rows = N - LEAF_START
    rb = 2000
    grid = rows // rb
    off = LEAF_START // rb
    h50, c50 = pl.pallas_call(
        _init_body,
        grid=(grid,),
        in_specs=[pl.BlockSpec((rb, X), lambda i: (i + 25, 0)),
                  pl.BlockSpec((X, 2 * H), lambda i: (0, 0)),
                  pl.BlockSpec((1, 2 * H), lambda i: (0, 0))],
        out_specs=[pl.BlockSpec((rb, H), lambda i: (i, 0)),
                   pl.BlockSpec((rb, H), lambda i: (i, 0))],
        out_shape=[jax.ShapeDtypeStruct((rows, H), jnp.float32),
                   jax.ShapeDtypeStruct((rows, H), jnp.float32)],
    )(x, W_w, wb)

    ch_rows = N - L16_START
    pad = 2 * P15_PAD - ch_rows
    hch = jnp.pad(h50[L15_LEAVES:], ((0, pad), (0, 0)))
    cch = jnp.pad(c50[L15_LEAVES:], ((0, pad), (0, 0)))
    hcat15 = hch.reshape(P15_PAD, 2 * H)
    ccat15 = cch.reshape(P15_PAD, 2 * H)
    hp15, cp15 = _level_call(hcat15, ccat15, Uf_w, ufb, Uiou_w, uiob)

    h15 = jnp.concatenate([hp15[:P15], h50[:L15_LEAVES]], axis=0)
    c_lvl = jnp.concatenate([cp15[:P15], c50[:L15_LEAVES]], axis=0)
    h_lvl = h15

    level_h = []
    for L in range(14, -1, -1):
        P = 1 << L
        hcat = h_lvl.reshape(P, 2 * H)
        ccat = c_lvl.reshape(P, 2 * H)
        if P < 8:
            hcat = jnp.pad(hcat, ((0, 8 - P), (0, 0)))
            ccat = jnp.pad(ccat, ((0, 8 - P), (0, 0)))
        h_lvl, c_lvl = _level_call(hcat, ccat, Uf_w, ufb, Uiou_w, uiob)
        h_lvl, c_lvl = h_lvl[:P], c_lvl[:P]
        level_h.append(h_lvl)

    parts = list(reversed(level_h))
    parts.append(h15)
    parts.append(h50[L15_LEAVES:])
    return jnp.concatenate(parts, axis=0)
